# baseline (speedup 1.0000x reference)
import numpy as np
import jax
import jax.numpy as jnp
from jax import lax
from jax.experimental import pallas as pl
from jax.experimental.pallas import tpu as pltpu
from jax.experimental.pallas import tpu_sc as plsc

_B, _C, _H, _W = 4, 192, 224, 224

_SWAPPED = [
    2, 3, 4, 5, 6, 7, 8, 10, 11, 15, 16, 18, 19, 20, 22, 24, 29, 30, 31, 32,
    34, 35, 37, 39, 42, 43, 44, 45, 49, 50, 53, 54, 56, 58, 61, 63, 65, 67,
    69, 70, 72, 77, 78, 80, 81, 82, 83, 85, 90, 92, 94, 96, 99, 101, 102,
    108, 110, 111, 112, 114, 117, 118, 121, 123, 129, 130, 137, 138, 139,
    140, 142, 144, 147, 148, 152, 153, 155, 156, 157, 159, 163, 167, 169,
    173, 174, 175, 176, 177, 178, 179, 183, 184, 185, 186, 188, 189,
]
_MASK = np.zeros(_C, dtype=bool)
_MASK[np.asarray(_SWAPPED)] = True

_MASK_WORDS = [
    int(np.uint32(sum(int(_MASK[w * 32 + i]) << i for i in range(32))).astype(np.int32))
    for w in range(_C // 32)
]

_NPLANE = _B * _C
_NW = 32
_PER_W = _NPLANE // _NW


def _swap_bit(c):
    q = c // 32
    r = c % 32
    word = jnp.int32(_MASK_WORDS[0])
    for k in range(1, len(_MASK_WORDS)):
        word = jnp.where(q == k, jnp.int32(_MASK_WORDS[k]), word)
    return lax.shift_right_logical(word, r) & 1


def _sc_body(f1p, f2p, out, buf0, buf1, sem0, sem1):
    w = lax.axis_index("s") * 2 + lax.axis_index("c")
    base = w * _PER_W
    bufs = ((buf0, sem0), (buf1, sem1))

    for phase, src_hbm in enumerate((f1p, f2p)):

        def dst_plane(j):
            p = base + j
            b = p // _C
            c = p % _C
            sw = _swap_bit(c)
            half = sw if phase == 0 else 1 - sw
            return (b * 2 * _C) + half * _C + c

        def start(j, par):
            buf, sem = bufs[par]
            return pltpu.async_copy(src_hbm.at[pl.ds(base + j, 1)], buf, sem)

        def drain(j, par):
            buf, sem = bufs[par]
            pltpu.async_copy(buf, out.at[pl.ds(dst_plane(j), 1)], sem).wait()

        cp = start(0, 0)
        for j in range(_PER_W):
            cp.wait()
            if j + 1 < _PER_W:
                nxt = start(j + 1, (j + 1) % 2)
            drain(j, j % 2)
            if j + 1 < _PER_W:
                cp = nxt


def kernel(f1, f2):
    B, C, H, W = f1.shape
    a = f1.reshape(_NPLANE, _H, _W)
    b = f2.reshape(_NPLANE, _H, _W)

    mesh = plsc.VectorSubcoreMesh(core_axis_name="c", subcore_axis_name="s")
    run = pl.kernel(
        _sc_body,
        mesh=mesh,
        out_type=jax.ShapeDtypeStruct((2 * _NPLANE, _H, _W), f1.dtype),
        scratch_types=[
            pltpu.VMEM((1, _H, _W), jnp.float32),
            pltpu.VMEM((1, _H, _W), jnp.float32),
            pltpu.SemaphoreType.DMA,
            pltpu.SemaphoreType.DMA,
        ],
        compiler_params=pltpu.CompilerParams(use_tc_tiling_on_sc=True),
    )
    out = run(a, b)
    return out.reshape(B, 2 * C, H, W)
